# baseline (speedup 1.0000x reference)
import functools

import jax
import jax.numpy as jnp
from jax import lax
from jax.experimental import pallas as pl
from jax.experimental.pallas import tpu as pltpu
from jax.experimental.pallas import tpu_sc as plsc

N = 100000
D = 128
G = 512

C = 128
FULL_CHUNKS = N // C
TAIL = N - FULL_CHUNKS * C
NW = 32
NBUF = 5
ROUNDS = 5
CPW = NBUF * ROUNDS
ROWS_PER_TILE = G // 16

_mesh = plsc.VectorSubcoreMesh(core_axis_name="c", subcore_axis_name="s")

_scratch = (
    [pltpu.VMEM((C, D), jnp.float32) for _ in range(NBUF)]
    + [pltpu.VMEM((CPW, C), jnp.int32),
       pltpu.VMEM((TAIL,), jnp.int32),
       pltpu.VMEM((TAIL, D), jnp.float32),
       pltpu.VMEM((ROWS_PER_TILE, D), jnp.float32),
       pltpu.VMEM_SHARED((G, D), jnp.float32)]
    + [pltpu.SemaphoreType.DMA for _ in range(2 * NBUF + 1)]
)


@functools.partial(
    pl.kernel,
    out_type=jax.ShapeDtypeStruct((2, G, D), jnp.float32),
    mesh=_mesh,
    scratch_types=_scratch,
)
def _sc_segment_sum(x_hbm, b_hbm, b3d_hbm, out_hbm, *refs):
    r_v = refs[0:NBUF]
    idx2d_v, tidx_v, trows_v, z_v, acc_sh = refs[NBUF:NBUF + 5]
    rsem = refs[NBUF + 5:NBUF + 5 + NBUF]
    ssem = refs[NBUF + 5 + NBUF:NBUF + 5 + 2 * NBUF]
    isem = refs[NBUF + 5 + 2 * NBUF]

    cid = lax.axis_index("c")
    sid = lax.axis_index("s")
    wid = cid * 16 + sid
    g0 = wid * CPW

    def valid(c):
        return g0 + c < FULL_CHUNKS

    def load(c, b):
        @pl.when(valid(c))
        def _():
            base = (g0 + c) * C
            pltpu.async_copy(x_hbm.at[pl.ds(base, C)], r_v[b], rsem[b])

    def process(c, b):
        @pl.when(valid(c))
        def _():
            base = (g0 + c) * C
            pltpu.make_async_copy(x_hbm.at[pl.ds(base, C)], r_v[b],
                                  rsem[b]).wait()
            pltpu.async_copy(r_v[b], acc_sh.at[idx2d_v.at[c]], ssem[b],
                             add=True)

    def drain(c, b):
        @pl.when(valid(c))
        def _():
            pltpu.make_async_copy(r_v[b], acc_sh.at[idx2d_v.at[c]],
                                  ssem[b]).wait()

    pltpu.async_copy(b3d_hbm.at[wid], idx2d_v, isem)
    for b in range(NBUF):
        load(b, b)

    for j in range(ROWS_PER_TILE):
        for k in range(D // 16):
            z_v[j, pl.ds(k * 16, 16)] = jnp.zeros((16,), jnp.float32)
    pltpu.sync_copy(z_v, acc_sh.at[pl.ds(sid * ROWS_PER_TILE, ROWS_PER_TILE)])
    pltpu.make_async_copy(b3d_hbm.at[wid], idx2d_v, isem).wait()
    plsc.subcore_barrier()

    def round_body(r, carry):
        for b in range(NBUF):
            process(NBUF * r + b, b)
        for b in range(NBUF):
            @pl.when(r < ROUNDS - 1)
            def _():
                drain(NBUF * r + b, b)
                load(NBUF * (r + 1) + b, b)
        return carry

    lax.fori_loop(0, ROUNDS, round_body, 0)
    for b in range(NBUF):
        drain(NBUF * (ROUNDS - 1) + b, b)

    @pl.when(wid == NW - 1)
    def _():
        tbase = FULL_CHUNKS * C
        pltpu.sync_copy(b_hbm.at[pl.ds(tbase, TAIL)], tidx_v)
        pltpu.sync_copy(x_hbm.at[pl.ds(tbase, TAIL)], trows_v)
        pltpu.sync_copy(trows_v, acc_sh.at[tidx_v], add=True)

    plsc.subcore_barrier()

    pltpu.sync_copy(
        acc_sh.at[pl.ds(sid * ROWS_PER_TILE, ROWS_PER_TILE)],
        out_hbm.at[cid, pl.ds(sid * ROWS_PER_TILE, ROWS_PER_TILE)])


def _combine_body(p_ref, o_ref):
    o_ref[...] = p_ref[0] + p_ref[1]


_combine = pl.pallas_call(
    _combine_body,
    out_shape=jax.ShapeDtypeStruct((G, D), jnp.float32),
)


def kernel(input, batch, num_graphs):
    b = batch.astype(jnp.int32)
    b3d = jnp.pad(b, (0, NW * CPW * C - N)).reshape(NW, CPW, C)
    partials = _sc_segment_sum(input, b, b3d)
    return _combine(partials)


# speedup vs baseline: 1.0067x; 1.0067x over previous
import functools

import jax
import jax.numpy as jnp
from jax import lax
from jax.experimental import pallas as pl
from jax.experimental.pallas import tpu as pltpu
from jax.experimental.pallas import tpu_sc as plsc

N = 100000
D = 128
G = 512

C = 128
FULL_CHUNKS = N // C
TAIL = N - FULL_CHUNKS * C
NW = 32
NBUF = 5
ROUNDS = 5
CPW = NBUF * ROUNDS
ROWS_PER_TILE = G // 16

_mesh = plsc.VectorSubcoreMesh(core_axis_name="c", subcore_axis_name="s")

_scratch = (
    [pltpu.VMEM((C, D), jnp.float32) for _ in range(NBUF)]
    + [pltpu.VMEM((C,), jnp.int32) for _ in range(NBUF)]
    + [pltpu.VMEM((TAIL,), jnp.int32),
       pltpu.VMEM((TAIL, D), jnp.float32),
       pltpu.VMEM((ROWS_PER_TILE, D), jnp.float32),
       pltpu.VMEM_SHARED((G, D), jnp.float32)]
    + [pltpu.SemaphoreType.DMA for _ in range(3 * NBUF)]
)


@functools.partial(
    pl.kernel,
    out_type=jax.ShapeDtypeStruct((2, G, D), jnp.float32),
    mesh=_mesh,
    scratch_types=_scratch,
)
def _sc_segment_sum(x_hbm, b_hbm, out_hbm, *refs):
    r_v = refs[0:NBUF]
    i_v = refs[NBUF:2 * NBUF]
    tidx_v, trows_v, z_v, acc_sh = refs[2 * NBUF:2 * NBUF + 4]
    rsem = refs[2 * NBUF + 4:2 * NBUF + 4 + NBUF]
    isem = refs[2 * NBUF + 4 + NBUF:2 * NBUF + 4 + 2 * NBUF]
    ssem = refs[2 * NBUF + 4 + 2 * NBUF:]

    cid = lax.axis_index("c")
    sid = lax.axis_index("s")
    wid = cid * 16 + sid
    g0 = wid * CPW

    def valid(c):
        return g0 + c < FULL_CHUNKS

    def load(c, b):
        @pl.when(valid(c))
        def _():
            base = (g0 + c) * C
            pltpu.async_copy(b_hbm.at[pl.ds(base, C)], i_v[b], isem[b])
            pltpu.async_copy(x_hbm.at[pl.ds(base, C)], r_v[b], rsem[b])

    def process(c, b):
        @pl.when(valid(c))
        def _():
            base = (g0 + c) * C
            pltpu.make_async_copy(b_hbm.at[pl.ds(base, C)], i_v[b],
                                  isem[b]).wait()
            pltpu.make_async_copy(x_hbm.at[pl.ds(base, C)], r_v[b],
                                  rsem[b]).wait()
            pltpu.async_copy(r_v[b], acc_sh.at[i_v[b]], ssem[b], add=True)

    def drain(c, b):
        @pl.when(valid(c))
        def _():
            pltpu.make_async_copy(r_v[b], acc_sh.at[i_v[b]], ssem[b]).wait()

    for b in range(NBUF):
        load(b, b)

    for j in range(ROWS_PER_TILE):
        for k in range(D // 16):
            z_v[j, pl.ds(k * 16, 16)] = jnp.zeros((16,), jnp.float32)
    pltpu.sync_copy(z_v, acc_sh.at[pl.ds(sid * ROWS_PER_TILE, ROWS_PER_TILE)])
    plsc.subcore_barrier()

    def round_body(r, carry):
        for b in range(NBUF):
            process(NBUF * r + b, b)
        for b in range(NBUF):
            @pl.when(r < ROUNDS - 1)
            def _():
                drain(NBUF * r + b, b)
                load(NBUF * (r + 1) + b, b)
        return carry

    lax.fori_loop(0, ROUNDS, round_body, 0)
    for b in range(NBUF):
        drain(NBUF * (ROUNDS - 1) + b, b)

    @pl.when(wid == NW - 1)
    def _():
        tbase = FULL_CHUNKS * C
        pltpu.sync_copy(b_hbm.at[pl.ds(tbase, TAIL)], tidx_v)
        pltpu.sync_copy(x_hbm.at[pl.ds(tbase, TAIL)], trows_v)
        pltpu.sync_copy(trows_v, acc_sh.at[tidx_v], add=True)

    plsc.subcore_barrier()

    pltpu.sync_copy(
        acc_sh.at[pl.ds(sid * ROWS_PER_TILE, ROWS_PER_TILE)],
        out_hbm.at[cid, pl.ds(sid * ROWS_PER_TILE, ROWS_PER_TILE)])


def _combine_body(p_ref, o_ref):
    o_ref[...] = p_ref[0] + p_ref[1]


_combine = pl.pallas_call(
    _combine_body,
    out_shape=jax.ShapeDtypeStruct((G, D), jnp.float32),
)


def kernel(input, batch, num_graphs):
    partials = _sc_segment_sum(input, batch.astype(jnp.int32))
    return _combine(partials)
